# baseline (speedup 1.0000x reference)
import functools

import jax
import jax.numpy as jnp
from jax import lax
from jax.experimental import pallas as pl
from jax.experimental.pallas import tpu as pltpu
from jax.experimental.pallas import tpu_sc as plsc

D = 64
SCALE = 8.0
NC, NS = 2, 16
NW = NC * NS
BLK = 128
LANES = 16
DT, DS = D // 8, 8


@functools.partial(jax.jit, static_argnums=(2, 3))
def _gather_scale(x_r, lut, n_s, n_b):
    n_steps = n_s
    assert n_steps >= 6 and n_steps % 2 == 0

    mesh = plsc.VectorSubcoreMesh(core_axis_name="c", subcore_axis_name="s")

    @functools.partial(
        pl.kernel,
        mesh=mesh,
        compiler_params=pltpu.CompilerParams(
            use_tc_tiling_on_sc=False, needs_layout_passes=False
        ),
        out_type=jax.ShapeDtypeStruct((n_s, DT, NW, DS * BLK), jnp.float32),
        scratch_types=[
            pltpu.VMEM((n_s, BLK), jnp.int32),
            pltpu.VMEM((BLK, D), jnp.float32),
            pltpu.VMEM((BLK, D), jnp.float32),
            pltpu.VMEM((DT * DS * BLK,), jnp.float32),
            pltpu.VMEM((DT * DS * BLK,), jnp.float32),
            pltpu.SemaphoreType.DMA,
            pltpu.SemaphoreType.DMA,
            pltpu.SemaphoreType.DMA,
            pltpu.SemaphoreType.DMA,
        ],
    )
    def k(x_hbm, lut_hbm, out_hbm, idx_v, ga, gb, ta, tb, gsa, gsb, ssa, ssb):
        wid = lax.axis_index("s") * NC + lax.axis_index("c")
        gbuf, tbuf = (ga, gb), (ta, tb)
        gsem, ssem = (gsa, gsb), (ssa, ssb)

        pltpu.sync_copy(x_hbm.at[wid], idx_v)

        def fire_gather(s, b):
            pltpu.async_copy(lut_hbm.at[idx_v.at[s]], gbuf[b], gsem[b])

        def drain_gather(b):
            pltpu.make_async_copy(
                lut_hbm.at[pl.ds(0, BLK)], gbuf[b], gsem[b]
            ).wait()

        def fire_store(s, b):
            for dt in range(DT):
                pltpu.async_copy(
                    tbuf[b].at[pl.ds(dt * DS * BLK, DS * BLK)],
                    out_hbm.at[s, dt, wid], ssem[b],
                )

        def drain_store(b):
            for dt in range(DT):
                pltpu.make_async_copy(
                    tbuf[b].at[pl.ds(dt * DS * BLK, DS * BLK)],
                    out_hbm.at[0, dt, wid], ssem[b],
                ).wait()

        iota = lax.iota(jnp.int32, LANES)
        off = (iota // 8) * (DS * BLK) + (iota % 8) * BLK

        def scale_transpose(b):
            src, dst = gbuf[b], tbuf[b]

            @plsc.parallel_loop(0, BLK, unroll=2)
            def _(i):
                for j in range(D // LANES):
                    v = src[i, pl.ds(j * LANES, LANES)] * SCALE
                    plsc.store_scatter(dst, [off + (2 * j) * (DS * BLK) + i], v)

        fire_gather(0, 0)
        fire_gather(1, 1)
        for b in range(2):
            drain_gather(b)
            scale_transpose(b)
            fire_store(b, b)
            fire_gather(b + 2, b)

        def step(it, carry):
            s0 = 2 + 2 * it
            for b in range(2):
                s = s0 + b
                drain_gather(b)
                drain_store(b)
                scale_transpose(b)
                fire_store(s, b)
                fire_gather(s + 2, b)
            return carry

        lax.fori_loop(0, (n_steps - 4) // 2, step, 0)

        for b in range(2):
            drain_gather(b)
            drain_store(b)
            scale_transpose(b)
            fire_store(n_steps - 2 + b, b)
        for b in range(2):
            drain_store(b)

    return k(x_r, lut)


def kernel(x, lut):
    n_b, n_s = x.shape
    x_r = jnp.transpose(x.T.reshape(n_s, NW, BLK), (1, 0, 2))
    out5 = _gather_scale(x_r, lut, n_s, n_b)
    out = (out5.reshape(n_s, DT, NW, DS, BLK)
           .transpose(2, 4, 0, 1, 3).reshape(n_b, n_s, D))
    return out


# speedup vs baseline: 1.6268x; 1.6268x over previous
import functools

import jax
import jax.numpy as jnp
from jax import lax
from jax.experimental import pallas as pl
from jax.experimental.pallas import tpu as pltpu
from jax.experimental.pallas import tpu_sc as plsc

D = 64
SCALE = 8.0
NC, NS = 2, 16
NW = NC * NS
BLK = 128
LANES = 16
DT, DS = D // 8, 8


@functools.partial(jax.jit, static_argnums=(2, 3))
def _gather_scale(x_r, lut, n_s, n_b):
    n_steps = n_s
    assert n_steps >= 6 and n_steps % 2 == 0

    mesh = plsc.VectorSubcoreMesh(core_axis_name="c", subcore_axis_name="s")

    @functools.partial(
        pl.kernel,
        mesh=mesh,
        compiler_params=pltpu.CompilerParams(
            use_tc_tiling_on_sc=False, needs_layout_passes=False
        ),
        out_type=jax.ShapeDtypeStruct((n_s, DT, NW, DS * BLK), jnp.float32),
        scratch_types=[
            pltpu.VMEM((n_s, BLK), jnp.int32),
            pltpu.VMEM((BLK, D), jnp.float32),
            pltpu.VMEM((BLK, D), jnp.float32),
            pltpu.VMEM((DT * DS * BLK,), jnp.float32),
            pltpu.VMEM((DT * DS * BLK,), jnp.float32),
            pltpu.SemaphoreType.DMA,
            pltpu.SemaphoreType.DMA,
            pltpu.SemaphoreType.DMA,
            pltpu.SemaphoreType.DMA,
        ],
    )
    def k(x_hbm, lut_hbm, out_hbm, idx_v, ga, gb, ta, tb, gsa, gsb, ssa, ssb):
        wid = lax.axis_index("s") * NC + lax.axis_index("c")
        gbuf, tbuf = (ga, gb), (ta, tb)
        gsem, ssem = (gsa, gsb), (ssa, ssb)

        pltpu.sync_copy(x_hbm.at[wid], idx_v)

        def fire_gather(s, b):
            pltpu.async_copy(lut_hbm.at[idx_v.at[s]], gbuf[b], gsem[b])

        def drain_gather(b):
            pltpu.make_async_copy(
                lut_hbm.at[pl.ds(0, BLK)], gbuf[b], gsem[b]
            ).wait()

        def fire_store(s, b):
            for dt in range(DT):
                pltpu.async_copy(
                    tbuf[b].at[pl.ds(dt * DS * BLK, DS * BLK)],
                    out_hbm.at[s, dt, wid], ssem[b],
                )

        def drain_store(b):
            for dt in range(DT):
                pltpu.make_async_copy(
                    tbuf[b].at[pl.ds(dt * DS * BLK, DS * BLK)],
                    out_hbm.at[0, dt, wid], ssem[b],
                ).wait()

        iota = lax.iota(jnp.int32, LANES)

        def scale_transpose(b):
            src, dst = gbuf[b], tbuf[b]

            @plsc.parallel_loop(0, BLK, unroll=2)
            def _(t):
                rowv = (t // LANES) * LANES + iota
                base = (t % LANES + iota) % LANES
                for g in range(D // LANES):
                    colv = base + g * LANES
                    v = plsc.load_gather(src, [rowv, colv])
                    plsc.store_scatter(dst, [colv * BLK + rowv], v * SCALE)

        fire_gather(0, 0)
        fire_gather(1, 1)
        for b in range(2):
            drain_gather(b)
            scale_transpose(b)
            fire_store(b, b)
            fire_gather(b + 2, b)

        def step(it, carry):
            s0 = 2 + 2 * it
            for b in range(2):
                s = s0 + b
                drain_gather(b)
                drain_store(b)
                scale_transpose(b)
                fire_store(s, b)
                fire_gather(s + 2, b)
            return carry

        lax.fori_loop(0, (n_steps - 4) // 2, step, 0)

        for b in range(2):
            drain_gather(b)
            drain_store(b)
            scale_transpose(b)
            fire_store(n_steps - 2 + b, b)
        for b in range(2):
            drain_store(b)

    return k(x_r, lut)


def kernel(x, lut):
    n_b, n_s = x.shape
    x_r = jnp.transpose(x.T.reshape(n_s, NW, BLK), (1, 0, 2))
    out5 = _gather_scale(x_r, lut, n_s, n_b)
    out = (out5.reshape(n_s, DT, NW, DS, BLK)
           .transpose(2, 4, 0, 1, 3).reshape(n_b, n_s, D))
    return out
